# baseline (speedup 1.0000x reference)
import jax
import jax.numpy as jnp
from jax.experimental import pallas as pl
from jax.experimental.pallas import tpu as pltpu

_CHUNK = 512


def _matmul_body(x_hbm, w_ref, b_ref, o_hbm, xs, os, xsem, osem):
    m = x_hbm.shape[0]
    n_chunks = m // _CHUNK

    def xcopy(c):
        return pltpu.make_async_copy(
            x_hbm.at[pl.ds(c * _CHUNK, _CHUNK), :], xs.at[c % 2], xsem.at[c % 2]
        )

    def ocopy(c):
        return pltpu.make_async_copy(
            os.at[c % 2], o_hbm.at[pl.ds(c * _CHUNK, _CHUNK), :], osem.at[c % 2]
        )

    xcopy(0).start()
    xcopy(1).start()
    for c in range(n_chunks):
        xcopy(c).wait()
        if c >= 2:
            ocopy(c - 2).wait()
        os[c % 2] = (
            jax.lax.dot_general(
                xs[c % 2],
                w_ref[...],
                dimension_numbers=(((1,), (0,)), ((), ())),
                precision=jax.lax.Precision.DEFAULT,
                preferred_element_type=jnp.float32,
            )
            + b_ref[...]
        )
        ocopy(c).start()
        if c + 2 < n_chunks:
            xcopy(c + 2).start()
    ocopy(n_chunks - 2).wait()
    ocopy(n_chunks - 1).wait()


def kernel(inputs, W, b):
    m, k = inputs.shape
    n = W.shape[1]
    b2 = b.reshape(1, n)
    return pl.pallas_call(
        _matmul_body,
        grid=(1,),
        in_specs=[
            pl.BlockSpec(memory_space=pltpu.MemorySpace.HBM),
            pl.BlockSpec((k, n), lambda i: (0, 0)),
            pl.BlockSpec((1, n), lambda i: (0, 0)),
        ],
        out_specs=pl.BlockSpec(memory_space=pltpu.MemorySpace.HBM),
        out_shape=jax.ShapeDtypeStruct((m, n), jnp.float32),
        scratch_shapes=[
            pltpu.VMEM((2, _CHUNK, k), jnp.float32),
            pltpu.VMEM((2, _CHUNK, n), jnp.float32),
            pltpu.SemaphoreType.DMA((2,)),
            pltpu.SemaphoreType.DMA((2,)),
        ],
        compiler_params=pltpu.CompilerParams(
            dimension_semantics=("arbitrary",),
        ),
    )(inputs, W, b2)


# speedup vs baseline: 1.0547x; 1.0547x over previous
import jax
import jax.numpy as jnp
from jax.experimental import pallas as pl
from jax.experimental.pallas import tpu as pltpu

_BM = 1024


def _matmul_body(x_ref, w_ref, b_ref, o_ref):
    o_ref[...] = (
        jax.lax.dot_general(
            x_ref[...],
            w_ref[...],
            dimension_numbers=(((1,), (0,)), ((), ())),
            precision=jax.lax.Precision.DEFAULT,
            preferred_element_type=jnp.float32,
        )
        + b_ref[...]
    )


def kernel(inputs, W, b):
    m, k = inputs.shape
    n = W.shape[1]
    b2 = b.reshape(1, n)
    grid = (m // _BM,)
    return pl.pallas_call(
        _matmul_body,
        grid=grid,
        in_specs=[
            pl.BlockSpec((_BM, k), lambda i: (i, 0)),
            pl.BlockSpec((k, n), lambda i: (0, 0)),
            pl.BlockSpec((1, n), lambda i: (0, 0)),
        ],
        out_specs=pl.BlockSpec((_BM, n), lambda i: (i, 0)),
        out_shape=jax.ShapeDtypeStruct((m, n), jnp.float32),
        compiler_params=pltpu.CompilerParams(
            dimension_semantics=("arbitrary",),
        ),
    )(inputs, W, b2)
